# baseline (speedup 1.0000x reference)
import functools

import jax
import jax.numpy as jnp
from jax import lax
from jax.experimental import pallas as pl
from jax.experimental.pallas import tpu as pltpu
from jax.experimental.pallas import tpu_sc as plsc

N_NODES = 10000
N_EDGES = 320000
D = 128
N_CLASS = 10

NUM_CORES = 2
NUM_SUBCORES = 16
NUM_TILES = NUM_CORES * NUM_SUBCORES

CHUNK = 128
EDGE_ROWS = 2560
ROWS_PER_TILE = EDGE_ROWS // NUM_TILES
NBUF = 2
IDX_SEG = 40
AGG_ROWS = 10240
ROWS_PER_SUBCORE = AGG_ROWS // NUM_SUBCORES
DUMMY_ROW = N_NODES


def _sc_segment_sum(h, src2d, dst2d):
    mesh = plsc.VectorSubcoreMesh(core_axis_name="c", subcore_axis_name="s")

    @functools.partial(
        pl.kernel,
        mesh=mesh,
        out_type=jax.ShapeDtypeStruct((NUM_CORES, AGG_ROWS, D), jnp.float32),
        scratch_types=[
            pltpu.VMEM((IDX_SEG, CHUNK), jnp.int32),
            pltpu.VMEM((IDX_SEG, CHUNK), jnp.int32),
            pltpu.VMEM((NBUF, CHUNK, D), jnp.float32),
            pltpu.VMEM_SHARED((AGG_ROWS, D), jnp.float32),
            pltpu.SemaphoreType.DMA,
            pltpu.SemaphoreType.DMA,
        ],
    )
    def k(h_hbm, src_hbm, dst_hbm, out_hbm, src_v, dst_v, rows_v, agg_sh,
          sem0, sem1):
        c = lax.axis_index("c")
        s = lax.axis_index("s")
        tid = c * NUM_SUBCORES + s

        def zrow(i, carry):
            def zcol(j, carry2):
                rows_v[0, i, pl.ds(j * 16, 16)] = jnp.zeros((16,), jnp.float32)
                return carry2
            return lax.fori_loop(0, D // 16, zcol, carry)
        lax.fori_loop(0, CHUNK, zrow, 0)
        zbase = s * ROWS_PER_SUBCORE
        for z in range(ROWS_PER_SUBCORE // CHUNK):
            pltpu.sync_copy(rows_v.at[0], agg_sh.at[pl.ds(zbase + z * CHUNK, CHUNK)])
        plsc.subcore_barrier()

        sems = [sem0, sem1]
        for seg in range(ROWS_PER_TILE // IDX_SEG):
            ibase = tid * ROWS_PER_TILE + seg * IDX_SEG
            pltpu.sync_copy(src_hbm.at[pl.ds(ibase, IDX_SEG)], src_v)
            pltpu.sync_copy(dst_hbm.at[pl.ds(ibase, IDX_SEG)], dst_v)

            def body(i, carry):
                j = i * NBUF
                cds = [pltpu.async_copy(h_hbm.at[src_v.at[j + b]], rows_v.at[b],
                                        sems[b])
                       for b in range(NBUF)]
                for b in range(NBUF):
                    cds[b].wait()
                    pltpu.sync_copy(rows_v.at[b], agg_sh.at[dst_v.at[j + b]],
                                    add=True)
                return carry
            lax.fori_loop(0, IDX_SEG // NBUF, body, 0)
        plsc.subcore_barrier()

        obase = s * ROWS_PER_SUBCORE
        pltpu.sync_copy(agg_sh.at[pl.ds(obase, ROWS_PER_SUBCORE)],
                        out_hbm.at[c, pl.ds(obase, ROWS_PER_SUBCORE)])

    return k(h, src2d, dst2d)


def _tc_linear(x, Wt, b2d):
    def k(x_ref, w_ref, b_ref, o_ref):
        o_ref[...] = jnp.dot(x_ref[...], w_ref[...],
                             preferred_element_type=jnp.float32) + b_ref[...]
    return pl.pallas_call(
        k, out_shape=jax.ShapeDtypeStruct((N_NODES, D), jnp.float32),
    )(x, Wt, b2d)


def _combine_bn_leaky(p_ref, h_ref, g_ref, be_ref):
    t = p_ref[0, :N_NODES] + p_ref[1, :N_NODES] + h_ref[...]
    t = jnp.maximum(t, 0.0)
    mu = jnp.mean(t, axis=0, keepdims=True)
    var = jnp.mean((t - mu) * (t - mu), axis=0, keepdims=True)
    tn = g_ref[...] * (t - mu) / jnp.sqrt(var + 1e-5) + be_ref[...]
    return jnp.where(tn >= 0.0, tn, 0.1 * tn)


def _tc_mid(P, h, g2d, be2d, Wt, b2d):
    def k(p_ref, h_ref, g_ref, be_ref, w_ref, b_ref, o_ref):
        tl = _combine_bn_leaky(p_ref, h_ref, g_ref, be_ref)
        o_ref[...] = jnp.dot(tl, w_ref[...],
                             preferred_element_type=jnp.float32) + b_ref[...]
    return pl.pallas_call(
        k, out_shape=jax.ShapeDtypeStruct((N_NODES, D), jnp.float32),
    )(P, h, g2d, be2d, Wt, b2d)


def _tc_final(P, h, g2d, be2d, p2d, WgT, bg2d):
    def k(p_ref, h_ref, g_ref, be_ref, pw_ref, wg_ref, bg_ref, out_ref, yp_ref):
        tl = _combine_bn_leaky(p_ref, h_ref, g_ref, be_ref)
        pw = pw_ref[0, 0]
        xc = jnp.clip(tl, 0.0, 100.0)
        xp = jnp.exp(pw * jnp.log(xc))
        pool = jnp.mean(xp, axis=0, keepdims=True)
        pool = jnp.clip(pool, 0.0, 100.0)
        pool = jnp.exp(jnp.log(pool) / pw)
        logits = jnp.dot(pool, wg_ref[...],
                         preferred_element_type=jnp.float32) + bg_ref[...]
        out_ref[...] = logits
        mx = jnp.max(logits, axis=1, keepdims=True)
        ids = lax.broadcasted_iota(jnp.int32, (1, N_CLASS), 1)
        cand = jnp.where(logits >= mx, ids, N_CLASS)
        yp_ref[...] = jnp.min(cand, axis=1, keepdims=True)
    return pl.pallas_call(
        k,
        out_shape=(jax.ShapeDtypeStruct((1, N_CLASS), jnp.float32),
                   jax.ShapeDtypeStruct((1, 1), jnp.int32)),
    )(P, h, g2d, be2d, p2d, WgT, bg2d)


def kernel(x, edge_index, W1, b1, W2, b2, W3, b3, g1, be1, g2, be2, g3, be3, p, Wg, bg):
    ei = edge_index.astype(jnp.int32)
    pad = EDGE_ROWS * CHUNK - N_EDGES
    order = jnp.argsort(ei[0])
    src_s = ei[0][order]
    dst_s = ei[1][order]
    src2d = jnp.concatenate(
        [src_s, jnp.zeros((pad,), jnp.int32)]).reshape(EDGE_ROWS, CHUNK)
    dst2d = jnp.concatenate(
        [dst_s, jnp.full((pad,), DUMMY_ROW, jnp.int32)]).reshape(EDGE_ROWS, CHUNK)

    W1t, W2t, W3t, WgT = W1.T, W2.T, W3.T, Wg.T
    b1d, b2d, b3d = b1.reshape(1, D), b2.reshape(1, D), b3.reshape(1, D)
    g1d, g2_2d, g3d = g1.reshape(1, D), g2.reshape(1, D), g3.reshape(1, D)
    be1d, be2d, be3d = be1.reshape(1, D), be2.reshape(1, D), be3.reshape(1, D)
    bg2d = bg.reshape(1, N_CLASS)
    p2d = p.reshape(1, 1)

    h1 = _tc_linear(x, W1t, b1d)
    P1 = _sc_segment_sum(h1, src2d, dst2d)
    h2 = _tc_mid(P1, h1, g1d, be1d, W2t, b2d)
    P2 = _sc_segment_sum(h2, src2d, dst2d)
    h3 = _tc_mid(P2, h2, g2_2d, be2d, W3t, b3d)
    P3 = _sc_segment_sum(h3, src2d, dst2d)
    output, yp = _tc_final(P3, h3, g3d, be3d, p2d, WgT, bg2d)
    return (output, yp.reshape(1))


# speedup vs baseline: 1.0224x; 1.0224x over previous
import functools

import jax
import jax.numpy as jnp
from jax import lax
from jax.experimental import pallas as pl
from jax.experimental.pallas import tpu as pltpu
from jax.experimental.pallas import tpu_sc as plsc

N_NODES = 10000
N_EDGES = 320000
D = 128
N_CLASS = 10

NUM_CORES = 2
NUM_SUBCORES = 16
NUM_TILES = NUM_CORES * NUM_SUBCORES

CHUNK = 128
EDGE_ROWS = 2560
ROWS_PER_TILE = EDGE_ROWS // NUM_TILES
NBUF = 2
IDX_SEG = 40
AGG_ROWS = 10240
ROWS_PER_SUBCORE = AGG_ROWS // NUM_SUBCORES
DUMMY_ROW = N_NODES


def _sc_segment_sum(h, src2d, dst2d):
    mesh = plsc.VectorSubcoreMesh(core_axis_name="c", subcore_axis_name="s")

    @functools.partial(
        pl.kernel,
        mesh=mesh,
        out_type=jax.ShapeDtypeStruct((NUM_CORES, AGG_ROWS, D), jnp.float32),
        scratch_types=[
            pltpu.VMEM((IDX_SEG, CHUNK), jnp.int32),
            pltpu.VMEM((IDX_SEG, CHUNK), jnp.int32),
            pltpu.VMEM((NBUF, CHUNK, D), jnp.float32),
            pltpu.VMEM_SHARED((AGG_ROWS, D), jnp.float32),
            pltpu.SemaphoreType.DMA,
            pltpu.SemaphoreType.DMA,
        ],
    )
    def k(h_hbm, src_hbm, dst_hbm, out_hbm, src_v, dst_v, rows_v, agg_sh,
          sem0, sem1):
        c = lax.axis_index("c")
        s = lax.axis_index("s")
        tid = c * NUM_SUBCORES + s

        def zrow(i, carry):
            def zcol(j, carry2):
                rows_v[0, i, pl.ds(j * 16, 16)] = jnp.zeros((16,), jnp.float32)
                return carry2
            return lax.fori_loop(0, D // 16, zcol, carry)
        lax.fori_loop(0, CHUNK, zrow, 0)
        zbase = s * ROWS_PER_SUBCORE
        for z in range(ROWS_PER_SUBCORE // CHUNK):
            pltpu.sync_copy(rows_v.at[0], agg_sh.at[pl.ds(zbase + z * CHUNK, CHUNK)])
        plsc.subcore_barrier()

        sems = [sem0, sem1]
        for seg in range(ROWS_PER_TILE // IDX_SEG):
            ibase = tid * ROWS_PER_TILE + seg * IDX_SEG
            pltpu.sync_copy(src_hbm.at[pl.ds(ibase, IDX_SEG)], src_v)
            pltpu.sync_copy(dst_hbm.at[pl.ds(ibase, IDX_SEG)], dst_v)

            def body(i, carry):
                j = i * NBUF
                cds = [pltpu.async_copy(h_hbm.at[src_v.at[j + b]], rows_v.at[b],
                                        sems[b])
                       for b in range(NBUF)]
                for b in range(NBUF):
                    cds[b].wait()
                    pltpu.sync_copy(rows_v.at[b], agg_sh.at[dst_v.at[j + b]],
                                    add=True)
                return carry
            lax.fori_loop(0, IDX_SEG // NBUF, body, 0)
        plsc.subcore_barrier()

        obase = s * ROWS_PER_SUBCORE
        pltpu.sync_copy(agg_sh.at[pl.ds(obase, ROWS_PER_SUBCORE)],
                        out_hbm.at[c, pl.ds(obase, ROWS_PER_SUBCORE)])

    return k(h, src2d, dst2d)


def _tc_linear(x, Wt, b2d):
    def k(x_ref, w_ref, b_ref, o_ref):
        o_ref[...] = jnp.dot(x_ref[...], w_ref[...],
                             preferred_element_type=jnp.float32) + b_ref[...]
    return pl.pallas_call(
        k, out_shape=jax.ShapeDtypeStruct((N_NODES, D), jnp.float32),
    )(x, Wt, b2d)


def _combine_bn_leaky(p_ref, h_ref, g_ref, be_ref):
    t = p_ref[0, :N_NODES] + p_ref[1, :N_NODES] + h_ref[...]
    t = jnp.maximum(t, 0.0)
    mu = jnp.mean(t, axis=0, keepdims=True)
    var = jnp.mean((t - mu) * (t - mu), axis=0, keepdims=True)
    tn = g_ref[...] * (t - mu) / jnp.sqrt(var + 1e-5) + be_ref[...]
    return jnp.where(tn >= 0.0, tn, 0.1 * tn)


def _tc_mid(P, h, g2d, be2d, Wt, b2d):
    def k(p_ref, h_ref, g_ref, be_ref, w_ref, b_ref, o_ref):
        tl = _combine_bn_leaky(p_ref, h_ref, g_ref, be_ref)
        o_ref[...] = jnp.dot(tl, w_ref[...],
                             preferred_element_type=jnp.float32) + b_ref[...]
    return pl.pallas_call(
        k, out_shape=jax.ShapeDtypeStruct((N_NODES, D), jnp.float32),
    )(P, h, g2d, be2d, Wt, b2d)


def _tc_final(P, h, g2d, be2d, p2d, WgT, bg2d):
    def k(p_ref, h_ref, g_ref, be_ref, pw_ref, wg_ref, bg_ref, out_ref, yp_ref):
        tl = _combine_bn_leaky(p_ref, h_ref, g_ref, be_ref)
        pw = pw_ref[0, 0]
        xc = jnp.clip(tl, 0.0, 100.0)
        xp = jnp.exp(pw * jnp.log(xc))
        pool = jnp.mean(xp, axis=0, keepdims=True)
        pool = jnp.clip(pool, 0.0, 100.0)
        pool = jnp.exp(jnp.log(pool) / pw)
        logits = jnp.dot(pool, wg_ref[...],
                         preferred_element_type=jnp.float32) + bg_ref[...]
        out_ref[...] = logits
        mx = jnp.max(logits, axis=1, keepdims=True)
        ids = lax.broadcasted_iota(jnp.int32, (1, N_CLASS), 1)
        cand = jnp.where(logits >= mx, ids, N_CLASS)
        yp_ref[...] = jnp.min(cand, axis=1, keepdims=True)
    return pl.pallas_call(
        k,
        out_shape=(jax.ShapeDtypeStruct((1, N_CLASS), jnp.float32),
                   jax.ShapeDtypeStruct((1, 1), jnp.int32)),
    )(P, h, g2d, be2d, p2d, WgT, bg2d)


def kernel(x, edge_index, W1, b1, W2, b2, W3, b3, g1, be1, g2, be2, g3, be3, p, Wg, bg):
    ei = edge_index.astype(jnp.int32)
    pad = EDGE_ROWS * CHUNK - N_EDGES
    src_s, dst_s = lax.sort((ei[0], ei[1]), num_keys=1)
    src2d = jnp.concatenate(
        [src_s, jnp.zeros((pad,), jnp.int32)]).reshape(EDGE_ROWS, CHUNK)
    dst2d = jnp.concatenate(
        [dst_s, jnp.full((pad,), DUMMY_ROW, jnp.int32)]).reshape(EDGE_ROWS, CHUNK)

    W1t, W2t, W3t, WgT = W1.T, W2.T, W3.T, Wg.T
    b1d, b2d, b3d = b1.reshape(1, D), b2.reshape(1, D), b3.reshape(1, D)
    g1d, g2_2d, g3d = g1.reshape(1, D), g2.reshape(1, D), g3.reshape(1, D)
    be1d, be2d, be3d = be1.reshape(1, D), be2.reshape(1, D), be3.reshape(1, D)
    bg2d = bg.reshape(1, N_CLASS)
    p2d = p.reshape(1, 1)

    h1 = _tc_linear(x, W1t, b1d)
    P1 = _sc_segment_sum(h1, src2d, dst2d)
    h2 = _tc_mid(P1, h1, g1d, be1d, W2t, b2d)
    P2 = _sc_segment_sum(h2, src2d, dst2d)
    h3 = _tc_mid(P2, h2, g2_2d, be2d, W3t, b3d)
    P3 = _sc_segment_sum(h3, src2d, dst2d)
    output, yp = _tc_final(P3, h3, g3d, be3d, p2d, WgT, bg2d)
    return (output, yp.reshape(1))


# speedup vs baseline: 1.3946x; 1.3641x over previous
import functools

import jax
import jax.numpy as jnp
from jax import lax
from jax.experimental import pallas as pl
from jax.experimental.pallas import tpu as pltpu
from jax.experimental.pallas import tpu_sc as plsc

N_NODES = 10000
N_EDGES = 320000
D = 128
N_CLASS = 10

NUM_CORES = 2
NUM_SUBCORES = 16
NUM_TILES = NUM_CORES * NUM_SUBCORES

CHUNK = 128
EDGE_ROWS = 2560
ROWS_PER_TILE = EDGE_ROWS // NUM_TILES
NBUF = 2
IDX_SEG = 16
WIN = 16
AGG_ROWS = 10240
ROWS_PER_SUBCORE = AGG_ROWS // NUM_SUBCORES
DUMMY_ROW = N_NODES


def _sc_segment_sum(h, src1d, dst2d, base1d, ok1d):
    mesh = plsc.VectorSubcoreMesh(core_axis_name="c", subcore_axis_name="s")

    @functools.partial(
        pl.kernel,
        mesh=mesh,
        out_type=jax.ShapeDtypeStruct((NUM_CORES, AGG_ROWS, D), jnp.float32),
        scratch_types=[
            pltpu.VMEM((IDX_SEG * CHUNK,), jnp.int32),
            pltpu.VMEM((IDX_SEG, CHUNK), jnp.int32),
            pltpu.VMEM((NBUF, CHUNK, D), jnp.float32),
            pltpu.VMEM((NBUF, WIN, D), jnp.float32),
            pltpu.VMEM_SHARED((AGG_ROWS, D), jnp.float32),
            pltpu.VMEM_SHARED((NUM_SUBCORES * 2 * IDX_SEG,), jnp.int32),
            pltpu.VMEM_SHARED((NUM_SUBCORES * IDX_SEG * CHUNK,), jnp.int32),
            pltpu.VMEM((2 * IDX_SEG,), jnp.int32),
            pltpu.SMEM((2 * IDX_SEG,), jnp.int32),
            pltpu.SMEM((CHUNK,), jnp.int32),
            pltpu.SemaphoreType.DMA,
            pltpu.SemaphoreType.DMA,
            pltpu.SemaphoreType.DMA,
            pltpu.SemaphoreType.DMA,
        ],
    )
    def k(h_hbm, src_hbm, dst_hbm, base_hbm, ok_hbm, out_hbm, src_v, dst_v,
          rows_v, win_v, agg_sh, stage_sh, srcstage_sh, meta_v, meta_sm,
          src_sm, sem0, sem1, sem2, sem3):
        c = lax.axis_index("c")
        s = lax.axis_index("s")
        tid = c * NUM_SUBCORES + s

        def zrow(i, carry):
            def zcol(j, carry2):
                rows_v[0, i, pl.ds(j * 16, 16)] = jnp.zeros((16,), jnp.float32)
                return carry2
            return lax.fori_loop(0, D // 16, zcol, carry)
        lax.fori_loop(0, CHUNK, zrow, 0)
        zbase = s * ROWS_PER_SUBCORE
        for z in range(ROWS_PER_SUBCORE // CHUNK):
            pltpu.sync_copy(rows_v.at[0], agg_sh.at[pl.ds(zbase + z * CHUNK, CHUNK)])
        plsc.subcore_barrier()

        wsems = [sem0, sem1]
        ssems = [sem2, sem3]
        for seg in range(ROWS_PER_TILE // IDX_SEG):
            ibase = tid * ROWS_PER_TILE + seg * IDX_SEG
            pltpu.sync_copy(src_hbm.at[pl.ds(ibase * CHUNK, IDX_SEG * CHUNK)],
                            src_v)
            pltpu.sync_copy(dst_hbm.at[pl.ds(ibase, IDX_SEG)], dst_v)
            pltpu.sync_copy(src_v,
                            srcstage_sh.at[pl.ds(s * IDX_SEG * CHUNK,
                                                 IDX_SEG * CHUNK)])
            pltpu.sync_copy(base_hbm.at[pl.ds(ibase, IDX_SEG)],
                            meta_v.at[pl.ds(0, IDX_SEG)])
            pltpu.sync_copy(ok_hbm.at[pl.ds(ibase, IDX_SEG)],
                            meta_v.at[pl.ds(IDX_SEG, IDX_SEG)])
            pltpu.sync_copy(meta_v,
                            stage_sh.at[pl.ds(s * 2 * IDX_SEG, 2 * IDX_SEG)])
            pltpu.sync_copy(stage_sh.at[pl.ds(s * 2 * IDX_SEG, 2 * IDX_SEG)],
                            meta_sm)

            def body(i, carry):
                js = [i * NBUF + b for b in range(NBUF)]
                a8s = [pl.multiple_of(meta_sm[j], 8) for j in js]
                wcds = []
                for b in range(NBUF):
                    wcds.append(pltpu.async_copy(
                        h_hbm.at[pl.ds(a8s[b], WIN)], win_v.at[b], wsems[b]))
                scds = []
                for b in range(NBUF):
                    j = js[b]
                    a8 = a8s[b]
                    span_ok = meta_sm[IDX_SEG + j] > 0

                    @pl.when(span_ok)
                    def _fast():
                        pltpu.sync_copy(
                            srcstage_sh.at[pl.ds(
                                (s * IDX_SEG + j) * CHUNK, CHUNK)], src_sm)
                        wcds[b].wait()

                        def edge(e, carry2):
                            r = src_sm[e] - a8
                            for g in range(D // 16):
                                rows_v[b, e, pl.ds(g * 16, 16)] = (
                                    win_v[b, r, pl.ds(g * 16, 16)])
                            return carry2
                        lax.fori_loop(0, CHUNK, edge, 0)

                    @pl.when(jnp.logical_not(span_ok))
                    def _slow():
                        wcds[b].wait()
                        pltpu.async_copy(
                            h_hbm.at[src_v.at[pl.ds(j * CHUNK, CHUNK)]],
                            rows_v.at[b], wsems[b]).wait()

                    scds.append(pltpu.async_copy(
                        rows_v.at[b], agg_sh.at[dst_v.at[j]], ssems[b],
                        add=True))
                for b in range(NBUF):
                    scds[b].wait()
                return carry
            lax.fori_loop(0, IDX_SEG // NBUF, body, 0)
        plsc.subcore_barrier()

        obase = s * ROWS_PER_SUBCORE
        pltpu.sync_copy(agg_sh.at[pl.ds(obase, ROWS_PER_SUBCORE)],
                        out_hbm.at[c, pl.ds(obase, ROWS_PER_SUBCORE)])

    return k(h, src1d, dst2d, base1d, ok1d)


def _tc_linear(x, Wt, b2d):
    def k(x_ref, w_ref, b_ref, o_ref):
        o_ref[...] = jnp.dot(x_ref[...], w_ref[...],
                             preferred_element_type=jnp.float32) + b_ref[...]
    return pl.pallas_call(
        k, out_shape=jax.ShapeDtypeStruct((N_NODES, D), jnp.float32),
    )(x, Wt, b2d)


def _combine_bn_leaky(p_ref, h_ref, g_ref, be_ref):
    t = p_ref[0, :N_NODES] + p_ref[1, :N_NODES] + h_ref[...]
    t = jnp.maximum(t, 0.0)
    mu = jnp.mean(t, axis=0, keepdims=True)
    var = jnp.mean((t - mu) * (t - mu), axis=0, keepdims=True)
    tn = g_ref[...] * (t - mu) / jnp.sqrt(var + 1e-5) + be_ref[...]
    return jnp.where(tn >= 0.0, tn, 0.1 * tn)


def _tc_mid(P, h, g2d, be2d, Wt, b2d):
    def k(p_ref, h_ref, g_ref, be_ref, w_ref, b_ref, o_ref):
        tl = _combine_bn_leaky(p_ref, h_ref, g_ref, be_ref)
        o_ref[...] = jnp.dot(tl, w_ref[...],
                             preferred_element_type=jnp.float32) + b_ref[...]
    return pl.pallas_call(
        k, out_shape=jax.ShapeDtypeStruct((N_NODES, D), jnp.float32),
    )(P, h, g2d, be2d, Wt, b2d)


def _tc_final(P, h, g2d, be2d, p2d, WgT, bg2d):
    def k(p_ref, h_ref, g_ref, be_ref, pw_ref, wg_ref, bg_ref, out_ref, yp_ref):
        tl = _combine_bn_leaky(p_ref, h_ref, g_ref, be_ref)
        pw = pw_ref[0, 0]
        xc = jnp.clip(tl, 0.0, 100.0)
        xp = jnp.exp(pw * jnp.log(xc))
        pool = jnp.mean(xp, axis=0, keepdims=True)
        pool = jnp.clip(pool, 0.0, 100.0)
        pool = jnp.exp(jnp.log(pool) / pw)
        logits = jnp.dot(pool, wg_ref[...],
                         preferred_element_type=jnp.float32) + bg_ref[...]
        out_ref[...] = logits
        mx = jnp.max(logits, axis=1, keepdims=True)
        ids = lax.broadcasted_iota(jnp.int32, (1, N_CLASS), 1)
        cand = jnp.where(logits >= mx, ids, N_CLASS)
        yp_ref[...] = jnp.min(cand, axis=1, keepdims=True)
    return pl.pallas_call(
        k,
        out_shape=(jax.ShapeDtypeStruct((1, N_CLASS), jnp.float32),
                   jax.ShapeDtypeStruct((1, 1), jnp.int32)),
    )(P, h, g2d, be2d, p2d, WgT, bg2d)


def kernel(x, edge_index, W1, b1, W2, b2, W3, b3, g1, be1, g2, be2, g3, be3, p, Wg, bg):
    ei = edge_index.astype(jnp.int32)
    pad = EDGE_ROWS * CHUNK - N_EDGES
    src_s, dst_s = lax.sort((ei[0], ei[1]), num_keys=1)
    src1d = jnp.concatenate(
        [src_s, jnp.full((pad,), N_NODES - 1, jnp.int32)])
    s2 = src1d.reshape(EDGE_ROWS, CHUNK)
    amin = s2[:, 0]
    amax = s2[:, CHUNK - 1]
    base1d = jnp.minimum((amin // 8) * 8, N_NODES - WIN)
    ok1d = (amax - base1d < WIN).astype(jnp.int32)
    dst2d = jnp.concatenate(
        [dst_s, jnp.full((pad,), DUMMY_ROW, jnp.int32)]).reshape(EDGE_ROWS, CHUNK)

    W1t, W2t, W3t, WgT = W1.T, W2.T, W3.T, Wg.T
    b1d, b2d, b3d = b1.reshape(1, D), b2.reshape(1, D), b3.reshape(1, D)
    g1d, g2_2d, g3d = g1.reshape(1, D), g2.reshape(1, D), g3.reshape(1, D)
    be1d, be2d, be3d = be1.reshape(1, D), be2.reshape(1, D), be3.reshape(1, D)
    bg2d = bg.reshape(1, N_CLASS)
    p2d = p.reshape(1, 1)

    h1 = _tc_linear(x, W1t, b1d)
    P1 = _sc_segment_sum(h1, src1d, dst2d, base1d, ok1d)
    h2 = _tc_mid(P1, h1, g1d, be1d, W2t, b2d)
    P2 = _sc_segment_sum(h2, src1d, dst2d, base1d, ok1d)
    h3 = _tc_mid(P2, h2, g2_2d, be2d, W3t, b3d)
    P3 = _sc_segment_sum(h3, src1d, dst2d, base1d, ok1d)
    output, yp = _tc_final(P3, h3, g3d, be3d, p2d, WgT, bg2d)
    return (output, yp.reshape(1))


# speedup vs baseline: 1.4873x; 1.0665x over previous
import functools

import jax
import jax.numpy as jnp
from jax import lax
from jax.experimental import pallas as pl
from jax.experimental.pallas import tpu as pltpu
from jax.experimental.pallas import tpu_sc as plsc

N_NODES = 10000
N_EDGES = 320000
D = 128
N_CLASS = 10

NUM_CORES = 2
NUM_SUBCORES = 16
NUM_TILES = NUM_CORES * NUM_SUBCORES

CHUNK = 128
EDGE_ROWS = 2560
ROWS_PER_TILE = EDGE_ROWS // NUM_TILES
NBUF = 2
IDX_SEG = 16
WIN = 16
AGG_ROWS = 10240
ROWS_PER_SUBCORE = AGG_ROWS // NUM_SUBCORES
DUMMY_ROW = N_NODES


def _sc_segment_sum(h, src1d, dst2d, base1d, ok1d):
    mesh = plsc.VectorSubcoreMesh(core_axis_name="c", subcore_axis_name="s")

    @functools.partial(
        pl.kernel,
        mesh=mesh,
        out_type=jax.ShapeDtypeStruct((NUM_CORES, AGG_ROWS, D), jnp.float32),
        scratch_types=[
            pltpu.VMEM((IDX_SEG * CHUNK,), jnp.int32),
            pltpu.VMEM((IDX_SEG, CHUNK), jnp.int32),
            pltpu.VMEM((NBUF, CHUNK, D), jnp.float32),
            pltpu.VMEM((NBUF, WIN, D), jnp.float32),
            pltpu.VMEM_SHARED((AGG_ROWS, D), jnp.float32),
            pltpu.VMEM_SHARED((NUM_SUBCORES * 2 * IDX_SEG,), jnp.int32),
            pltpu.VMEM_SHARED((NUM_SUBCORES * IDX_SEG * CHUNK,), jnp.int32),
            pltpu.VMEM((2 * IDX_SEG,), jnp.int32),
            pltpu.SMEM((2 * IDX_SEG,), jnp.int32),
            pltpu.SMEM((CHUNK,), jnp.int32),
            pltpu.SMEM((CHUNK,), jnp.int32),
            pltpu.SemaphoreType.DMA,
            pltpu.SemaphoreType.DMA,
            pltpu.SemaphoreType.DMA,
            pltpu.SemaphoreType.DMA,
            pltpu.SemaphoreType.DMA,
            pltpu.SemaphoreType.DMA,
        ],
    )
    def k(h_hbm, src_hbm, dst_hbm, base_hbm, ok_hbm, out_hbm, src_v, dst_v,
          rows_v, win_v, agg_sh, stage_sh, srcstage_sh, meta_v, meta_sm,
          src_sm0, src_sm1, sem0, sem1, sem2, sem3, sem4, sem5):
        c = lax.axis_index("c")
        s = lax.axis_index("s")
        tid = c * NUM_SUBCORES + s

        def zrow(i, carry):
            def zcol(j, carry2):
                rows_v[0, i, pl.ds(j * 16, 16)] = jnp.zeros((16,), jnp.float32)
                return carry2
            return lax.fori_loop(0, D // 16, zcol, carry)
        lax.fori_loop(0, CHUNK, zrow, 0)
        zbase = s * ROWS_PER_SUBCORE
        for z in range(ROWS_PER_SUBCORE // CHUNK):
            pltpu.sync_copy(rows_v.at[0], agg_sh.at[pl.ds(zbase + z * CHUNK, CHUNK)])
        plsc.subcore_barrier()

        wsems = [sem0, sem1]
        ssems = [sem2, sem3]
        for seg in range(ROWS_PER_TILE // IDX_SEG):
            ibase = tid * ROWS_PER_TILE + seg * IDX_SEG
            pltpu.sync_copy(src_hbm.at[pl.ds(ibase * CHUNK, IDX_SEG * CHUNK)],
                            src_v)
            pltpu.sync_copy(dst_hbm.at[pl.ds(ibase, IDX_SEG)], dst_v)
            pltpu.sync_copy(src_v,
                            srcstage_sh.at[pl.ds(s * IDX_SEG * CHUNK,
                                                 IDX_SEG * CHUNK)])
            pltpu.sync_copy(base_hbm.at[pl.ds(ibase, IDX_SEG)],
                            meta_v.at[pl.ds(0, IDX_SEG)])
            pltpu.sync_copy(ok_hbm.at[pl.ds(ibase, IDX_SEG)],
                            meta_v.at[pl.ds(IDX_SEG, IDX_SEG)])
            pltpu.sync_copy(meta_v,
                            stage_sh.at[pl.ds(s * 2 * IDX_SEG, 2 * IDX_SEG)])
            pltpu.sync_copy(stage_sh.at[pl.ds(s * 2 * IDX_SEG, 2 * IDX_SEG)],
                            meta_sm)

            fsems = [sem4, sem5]
            src_sms = [src_sm0, src_sm1]

            def fire_fill(j, b):
                return pltpu.async_copy(
                    srcstage_sh.at[pl.ds((s * IDX_SEG + j) * CHUNK, CHUNK)],
                    src_sms[b], fsems[b])

            def fire_win(j, b):
                a8 = pl.multiple_of(meta_sm[j], 8)
                return pltpu.async_copy(
                    h_hbm.at[pl.ds(a8, WIN)], win_v.at[b], wsems[b])

            for b in range(NBUF):
                fire_fill(b, b)
                fire_win(b, b)

            def body(i, carry):
                scds = []
                for b in range(NBUF):
                    j = i * NBUF + b
                    a8 = pl.multiple_of(meta_sm[j], 8)
                    span_ok = meta_sm[IDX_SEG + j] > 0
                    pltpu.make_async_copy(
                        srcstage_sh.at[pl.ds((s * IDX_SEG + j) * CHUNK,
                                             CHUNK)],
                        src_sms[b], fsems[b]).wait()

                    @pl.when(span_ok)
                    def _fast():
                        pltpu.make_async_copy(
                            h_hbm.at[pl.ds(a8, WIN)], win_v.at[b],
                            wsems[b]).wait()

                        def edge(m, carry2):
                            for u in range(2):
                                e = m * 2 + u
                                r = src_sms[b][e] - a8
                                for g in range(D // 16):
                                    rows_v[b, e, pl.ds(g * 16, 16)] = (
                                        win_v[b, r, pl.ds(g * 16, 16)])
                            return carry2
                        lax.fori_loop(0, CHUNK // 2, edge, 0)

                    @pl.when(jnp.logical_not(span_ok))
                    def _slow():
                        pltpu.make_async_copy(
                            h_hbm.at[pl.ds(a8, WIN)], win_v.at[b],
                            wsems[b]).wait()
                        pltpu.async_copy(
                            h_hbm.at[src_v.at[pl.ds(j * CHUNK, CHUNK)]],
                            rows_v.at[b], wsems[b]).wait()

                    scds.append(pltpu.async_copy(
                        rows_v.at[b], agg_sh.at[dst_v.at[j]], ssems[b],
                        add=True))

                    @pl.when(j + NBUF < IDX_SEG)
                    def _prefetch():
                        fire_fill(j + NBUF, b)
                        fire_win(j + NBUF, b)

                for b in range(NBUF):
                    scds[b].wait()
                return carry
            lax.fori_loop(0, IDX_SEG // NBUF, body, 0)
        plsc.subcore_barrier()

        obase = s * ROWS_PER_SUBCORE
        pltpu.sync_copy(agg_sh.at[pl.ds(obase, ROWS_PER_SUBCORE)],
                        out_hbm.at[c, pl.ds(obase, ROWS_PER_SUBCORE)])

    return k(h, src1d, dst2d, base1d, ok1d)


def _tc_linear(x, Wt, b2d):
    def k(x_ref, w_ref, b_ref, o_ref):
        o_ref[...] = jnp.dot(x_ref[...], w_ref[...],
                             preferred_element_type=jnp.float32) + b_ref[...]
    return pl.pallas_call(
        k, out_shape=jax.ShapeDtypeStruct((N_NODES, D), jnp.float32),
    )(x, Wt, b2d)


def _combine_bn_leaky(p_ref, h_ref, g_ref, be_ref):
    t = p_ref[0, :N_NODES] + p_ref[1, :N_NODES] + h_ref[...]
    t = jnp.maximum(t, 0.0)
    mu = jnp.mean(t, axis=0, keepdims=True)
    var = jnp.mean((t - mu) * (t - mu), axis=0, keepdims=True)
    tn = g_ref[...] * (t - mu) / jnp.sqrt(var + 1e-5) + be_ref[...]
    return jnp.where(tn >= 0.0, tn, 0.1 * tn)


def _tc_mid(P, h, g2d, be2d, Wt, b2d):
    def k(p_ref, h_ref, g_ref, be_ref, w_ref, b_ref, o_ref):
        tl = _combine_bn_leaky(p_ref, h_ref, g_ref, be_ref)
        o_ref[...] = jnp.dot(tl, w_ref[...],
                             preferred_element_type=jnp.float32) + b_ref[...]
    return pl.pallas_call(
        k, out_shape=jax.ShapeDtypeStruct((N_NODES, D), jnp.float32),
    )(P, h, g2d, be2d, Wt, b2d)


def _tc_final(P, h, g2d, be2d, p2d, WgT, bg2d):
    def k(p_ref, h_ref, g_ref, be_ref, pw_ref, wg_ref, bg_ref, out_ref, yp_ref):
        tl = _combine_bn_leaky(p_ref, h_ref, g_ref, be_ref)
        pw = pw_ref[0, 0]
        xc = jnp.clip(tl, 0.0, 100.0)
        xp = jnp.exp(pw * jnp.log(xc))
        pool = jnp.mean(xp, axis=0, keepdims=True)
        pool = jnp.clip(pool, 0.0, 100.0)
        pool = jnp.exp(jnp.log(pool) / pw)
        logits = jnp.dot(pool, wg_ref[...],
                         preferred_element_type=jnp.float32) + bg_ref[...]
        out_ref[...] = logits
        mx = jnp.max(logits, axis=1, keepdims=True)
        ids = lax.broadcasted_iota(jnp.int32, (1, N_CLASS), 1)
        cand = jnp.where(logits >= mx, ids, N_CLASS)
        yp_ref[...] = jnp.min(cand, axis=1, keepdims=True)
    return pl.pallas_call(
        k,
        out_shape=(jax.ShapeDtypeStruct((1, N_CLASS), jnp.float32),
                   jax.ShapeDtypeStruct((1, 1), jnp.int32)),
    )(P, h, g2d, be2d, p2d, WgT, bg2d)


def kernel(x, edge_index, W1, b1, W2, b2, W3, b3, g1, be1, g2, be2, g3, be3, p, Wg, bg):
    ei = edge_index.astype(jnp.int32)
    pad = EDGE_ROWS * CHUNK - N_EDGES
    src_s, dst_s = lax.sort((ei[0], ei[1]), num_keys=1)
    src1d = jnp.concatenate(
        [src_s, jnp.full((pad,), N_NODES - 1, jnp.int32)])
    s2 = src1d.reshape(EDGE_ROWS, CHUNK)
    amin = s2[:, 0]
    amax = s2[:, CHUNK - 1]
    base1d = jnp.minimum((amin // 8) * 8, N_NODES - WIN)
    ok1d = (amax - base1d < WIN).astype(jnp.int32)
    dst2d = jnp.concatenate(
        [dst_s, jnp.full((pad,), DUMMY_ROW, jnp.int32)]).reshape(EDGE_ROWS, CHUNK)

    W1t, W2t, W3t, WgT = W1.T, W2.T, W3.T, Wg.T
    b1d, b2d, b3d = b1.reshape(1, D), b2.reshape(1, D), b3.reshape(1, D)
    g1d, g2_2d, g3d = g1.reshape(1, D), g2.reshape(1, D), g3.reshape(1, D)
    be1d, be2d, be3d = be1.reshape(1, D), be2.reshape(1, D), be3.reshape(1, D)
    bg2d = bg.reshape(1, N_CLASS)
    p2d = p.reshape(1, 1)

    h1 = _tc_linear(x, W1t, b1d)
    P1 = _sc_segment_sum(h1, src1d, dst2d, base1d, ok1d)
    h2 = _tc_mid(P1, h1, g1d, be1d, W2t, b2d)
    P2 = _sc_segment_sum(h2, src1d, dst2d, base1d, ok1d)
    h3 = _tc_mid(P2, h2, g2_2d, be2d, W3t, b3d)
    P3 = _sc_segment_sum(h3, src1d, dst2d, base1d, ok1d)
    output, yp = _tc_final(P3, h3, g3d, be3d, p2d, WgT, bg2d)
    return (output, yp.reshape(1))


# speedup vs baseline: 2.6373x; 1.7732x over previous
import functools

import jax
import jax.numpy as jnp
from jax import lax
from jax.experimental import pallas as pl
from jax.experimental.pallas import tpu as pltpu
from jax.experimental.pallas import tpu_sc as plsc

N_NODES = 10000
N_EDGES = 320000
D = 128
N_CLASS = 10

NUM_CORES = 2
NUM_SUBCORES = 16
NUM_TILES = NUM_CORES * NUM_SUBCORES

CHUNK = 128
EDGE_ROWS = 2560
ROWS_PER_TILE = EDGE_ROWS // NUM_TILES
NBUF = 2
IDX_SEG = 16
WIN = 16
RUNS_W = WIN + 1
AGG_ROWS = 10240
ROWS_PER_SUBCORE = AGG_ROWS // NUM_SUBCORES
DUMMY_ROW = N_NODES


def _sc_segment_sum(h, src1d, dst2d, base1d, ok1d, runs1d):
    mesh = plsc.VectorSubcoreMesh(core_axis_name="c", subcore_axis_name="s")

    @functools.partial(
        pl.kernel,
        mesh=mesh,
        out_type=jax.ShapeDtypeStruct((NUM_CORES, AGG_ROWS, D), jnp.float32),
        scratch_types=[
            pltpu.VMEM((IDX_SEG * CHUNK,), jnp.int32),
            pltpu.VMEM((IDX_SEG, CHUNK), jnp.int32),
            pltpu.VMEM((NBUF, CHUNK, D), jnp.float32),
            pltpu.VMEM((NBUF, WIN, D), jnp.float32),
            pltpu.VMEM_SHARED((AGG_ROWS, D), jnp.float32),
            pltpu.VMEM_SHARED((NUM_SUBCORES * 2 * IDX_SEG,), jnp.int32),
            pltpu.VMEM_SHARED((NUM_SUBCORES * IDX_SEG * CHUNK,), jnp.int32),
            pltpu.VMEM_SHARED((NUM_SUBCORES * IDX_SEG * RUNS_W,), jnp.int32),
            pltpu.VMEM((2 * IDX_SEG,), jnp.int32),
            pltpu.VMEM((IDX_SEG * RUNS_W,), jnp.int32),
            pltpu.SMEM((2 * IDX_SEG,), jnp.int32),
            pltpu.SMEM((IDX_SEG * RUNS_W,), jnp.int32),
            pltpu.SMEM((CHUNK,), jnp.int32),
            pltpu.SMEM((CHUNK,), jnp.int32),
            pltpu.SemaphoreType.DMA,
            pltpu.SemaphoreType.DMA,
            pltpu.SemaphoreType.DMA,
            pltpu.SemaphoreType.DMA,
            pltpu.SemaphoreType.DMA,
            pltpu.SemaphoreType.DMA,
        ],
    )
    def k(h_hbm, src_hbm, dst_hbm, base_hbm, ok_hbm, runs_hbm, out_hbm,
          src_v, dst_v, rows_v, win_v, agg_sh, stage_sh, srcstage_sh,
          runstage_sh, meta_v, runs_v, meta_sm, runs_sm,
          src_sm0, src_sm1, sem0, sem1, sem2, sem3, sem4, sem5):
        c = lax.axis_index("c")
        s = lax.axis_index("s")
        tid = c * NUM_SUBCORES + s

        def zrow(i, carry):
            def zcol(j, carry2):
                rows_v[0, i, pl.ds(j * 16, 16)] = jnp.zeros((16,), jnp.float32)
                return carry2
            return lax.fori_loop(0, D // 16, zcol, carry)
        lax.fori_loop(0, CHUNK, zrow, 0)
        zbase = s * ROWS_PER_SUBCORE
        for z in range(ROWS_PER_SUBCORE // CHUNK):
            pltpu.sync_copy(rows_v.at[0], agg_sh.at[pl.ds(zbase + z * CHUNK, CHUNK)])
        plsc.subcore_barrier()

        wsems = [sem0, sem1]
        ssems = [sem2, sem3]
        for seg in range(ROWS_PER_TILE // IDX_SEG):
            ibase = tid * ROWS_PER_TILE + seg * IDX_SEG
            pltpu.sync_copy(src_hbm.at[pl.ds(ibase * CHUNK, IDX_SEG * CHUNK)],
                            src_v)
            pltpu.sync_copy(dst_hbm.at[pl.ds(ibase, IDX_SEG)], dst_v)
            pltpu.sync_copy(src_v,
                            srcstage_sh.at[pl.ds(s * IDX_SEG * CHUNK,
                                                 IDX_SEG * CHUNK)])
            pltpu.sync_copy(base_hbm.at[pl.ds(ibase, IDX_SEG)],
                            meta_v.at[pl.ds(0, IDX_SEG)])
            pltpu.sync_copy(ok_hbm.at[pl.ds(ibase, IDX_SEG)],
                            meta_v.at[pl.ds(IDX_SEG, IDX_SEG)])
            pltpu.sync_copy(meta_v,
                            stage_sh.at[pl.ds(s * 2 * IDX_SEG, 2 * IDX_SEG)])
            pltpu.sync_copy(stage_sh.at[pl.ds(s * 2 * IDX_SEG, 2 * IDX_SEG)],
                            meta_sm)
            pltpu.sync_copy(runs_hbm.at[pl.ds(ibase * RUNS_W,
                                              IDX_SEG * RUNS_W)], runs_v)
            pltpu.sync_copy(runs_v,
                            runstage_sh.at[pl.ds(s * IDX_SEG * RUNS_W,
                                                 IDX_SEG * RUNS_W)])
            pltpu.sync_copy(runstage_sh.at[pl.ds(s * IDX_SEG * RUNS_W,
                                                 IDX_SEG * RUNS_W)], runs_sm)

            fsems = [sem4, sem5]
            src_sms = [src_sm0, src_sm1]

            def fire_fill(j, b):
                return pltpu.async_copy(
                    srcstage_sh.at[pl.ds((s * IDX_SEG + j) * CHUNK, CHUNK)],
                    src_sms[b], fsems[b])

            def fire_win(j, b):
                a8 = pl.multiple_of(meta_sm[j], 8)
                return pltpu.async_copy(
                    h_hbm.at[pl.ds(a8, WIN)], win_v.at[b], wsems[b])

            for b in range(NBUF):
                fire_fill(b, b)
                fire_win(b, b)

            def body(i, carry):
                scds = []
                for b in range(NBUF):
                    j = i * NBUF + b
                    a8 = pl.multiple_of(meta_sm[j], 8)
                    span_ok = meta_sm[IDX_SEG + j] > 0
                    pltpu.make_async_copy(
                        srcstage_sh.at[pl.ds((s * IDX_SEG + j) * CHUNK,
                                             CHUNK)],
                        src_sms[b], fsems[b]).wait()

                    @pl.when(span_ok)
                    def _fast():
                        pltpu.make_async_copy(
                            h_hbm.at[pl.ds(a8, WIN)], win_v.at[b],
                            wsems[b]).wait()

                        def runk(kk, carry2):
                            st = runs_sm[j * RUNS_W + kk]
                            en = runs_sm[j * RUNS_W + kk + 1]
                            r = src_sms[b][jnp.minimum(st, CHUNK - 1)] - a8
                            row = [win_v[b, r, pl.ds(g * 16, 16)]
                                   for g in range(D // 16)]

                            def inner(e, c3):
                                for g in range(D // 16):
                                    rows_v[b, e, pl.ds(g * 16, 16)] = row[g]
                                return c3
                            lax.fori_loop(st, en, inner, 0)
                            return carry2
                        lax.fori_loop(0, WIN, runk, 0)

                    @pl.when(jnp.logical_not(span_ok))
                    def _slow():
                        pltpu.make_async_copy(
                            h_hbm.at[pl.ds(a8, WIN)], win_v.at[b],
                            wsems[b]).wait()
                        pltpu.async_copy(
                            h_hbm.at[src_v.at[pl.ds(j * CHUNK, CHUNK)]],
                            rows_v.at[b], wsems[b]).wait()

                    scds.append(pltpu.async_copy(
                        rows_v.at[b], agg_sh.at[dst_v.at[j]], ssems[b],
                        add=True))

                    @pl.when(j + NBUF < IDX_SEG)
                    def _prefetch():
                        fire_fill(j + NBUF, b)
                        fire_win(j + NBUF, b)

                for b in range(NBUF):
                    scds[b].wait()
                return carry
            lax.fori_loop(0, IDX_SEG // NBUF, body, 0)
        plsc.subcore_barrier()

        obase = s * ROWS_PER_SUBCORE
        pltpu.sync_copy(agg_sh.at[pl.ds(obase, ROWS_PER_SUBCORE)],
                        out_hbm.at[c, pl.ds(obase, ROWS_PER_SUBCORE)])

    return k(h, src1d, dst2d, base1d, ok1d, runs1d)


def _tc_linear(x, Wt, b2d):
    def k(x_ref, w_ref, b_ref, o_ref):
        o_ref[...] = jnp.dot(x_ref[...], w_ref[...],
                             preferred_element_type=jnp.float32) + b_ref[...]
    return pl.pallas_call(
        k, out_shape=jax.ShapeDtypeStruct((N_NODES, D), jnp.float32),
    )(x, Wt, b2d)


def _combine_bn_leaky(p_ref, h_ref, g_ref, be_ref):
    t = p_ref[0, :N_NODES] + p_ref[1, :N_NODES] + h_ref[...]
    t = jnp.maximum(t, 0.0)
    mu = jnp.mean(t, axis=0, keepdims=True)
    var = jnp.mean((t - mu) * (t - mu), axis=0, keepdims=True)
    tn = g_ref[...] * (t - mu) / jnp.sqrt(var + 1e-5) + be_ref[...]
    return jnp.where(tn >= 0.0, tn, 0.1 * tn)


def _tc_mid(P, h, g2d, be2d, Wt, b2d):
    def k(p_ref, h_ref, g_ref, be_ref, w_ref, b_ref, o_ref):
        tl = _combine_bn_leaky(p_ref, h_ref, g_ref, be_ref)
        o_ref[...] = jnp.dot(tl, w_ref[...],
                             preferred_element_type=jnp.float32) + b_ref[...]
    return pl.pallas_call(
        k, out_shape=jax.ShapeDtypeStruct((N_NODES, D), jnp.float32),
    )(P, h, g2d, be2d, Wt, b2d)


def _tc_final(P, h, g2d, be2d, p2d, WgT, bg2d):
    def k(p_ref, h_ref, g_ref, be_ref, pw_ref, wg_ref, bg_ref, out_ref, yp_ref):
        tl = _combine_bn_leaky(p_ref, h_ref, g_ref, be_ref)
        pw = pw_ref[0, 0]
        xc = jnp.clip(tl, 0.0, 100.0)
        xp = jnp.exp(pw * jnp.log(xc))
        pool = jnp.mean(xp, axis=0, keepdims=True)
        pool = jnp.clip(pool, 0.0, 100.0)
        pool = jnp.exp(jnp.log(pool) / pw)
        logits = jnp.dot(pool, wg_ref[...],
                         preferred_element_type=jnp.float32) + bg_ref[...]
        out_ref[...] = logits
        mx = jnp.max(logits, axis=1, keepdims=True)
        ids = lax.broadcasted_iota(jnp.int32, (1, N_CLASS), 1)
        cand = jnp.where(logits >= mx, ids, N_CLASS)
        yp_ref[...] = jnp.min(cand, axis=1, keepdims=True)
    return pl.pallas_call(
        k,
        out_shape=(jax.ShapeDtypeStruct((1, N_CLASS), jnp.float32),
                   jax.ShapeDtypeStruct((1, 1), jnp.int32)),
    )(P, h, g2d, be2d, p2d, WgT, bg2d)


def kernel(x, edge_index, W1, b1, W2, b2, W3, b3, g1, be1, g2, be2, g3, be3, p, Wg, bg):
    ei = edge_index.astype(jnp.int32)
    pad = EDGE_ROWS * CHUNK - N_EDGES
    src_s, dst_s = lax.sort((ei[0], ei[1]), num_keys=1)
    src1d = jnp.concatenate(
        [src_s, jnp.full((pad,), N_NODES - 1, jnp.int32)])
    s2 = src1d.reshape(EDGE_ROWS, CHUNK)
    amin = s2[:, 0]
    amax = s2[:, CHUNK - 1]
    base1d = jnp.minimum((amin // 8) * 8, N_NODES - WIN)
    ok1d = (amax - base1d < WIN).astype(jnp.int32)
    lane = jnp.arange(CHUNK, dtype=jnp.int32)[None, :]
    isstart = jnp.concatenate(
        [jnp.ones((EDGE_ROWS, 1), bool), s2[:, 1:] != s2[:, :-1]], axis=1)
    rank = jnp.cumsum(isstart.astype(jnp.int32), axis=1) - 1
    cand = jnp.where(isstart, lane, CHUNK)
    runs = [jnp.min(jnp.where(rank == k, cand, CHUNK), axis=1)
            for k in range(WIN)]
    runs.append(jnp.full((EDGE_ROWS,), CHUNK, jnp.int32))
    runs1d = jnp.stack(runs, axis=1).reshape(-1)
    dst2d = jnp.concatenate(
        [dst_s, jnp.full((pad,), DUMMY_ROW, jnp.int32)]).reshape(EDGE_ROWS, CHUNK)

    W1t, W2t, W3t, WgT = W1.T, W2.T, W3.T, Wg.T
    b1d, b2d, b3d = b1.reshape(1, D), b2.reshape(1, D), b3.reshape(1, D)
    g1d, g2_2d, g3d = g1.reshape(1, D), g2.reshape(1, D), g3.reshape(1, D)
    be1d, be2d, be3d = be1.reshape(1, D), be2.reshape(1, D), be3.reshape(1, D)
    bg2d = bg.reshape(1, N_CLASS)
    p2d = p.reshape(1, 1)

    h1 = _tc_linear(x, W1t, b1d)
    P1 = _sc_segment_sum(h1, src1d, dst2d, base1d, ok1d, runs1d)
    h2 = _tc_mid(P1, h1, g1d, be1d, W2t, b2d)
    P2 = _sc_segment_sum(h2, src1d, dst2d, base1d, ok1d, runs1d)
    h3 = _tc_mid(P2, h2, g2_2d, be2d, W3t, b3d)
    P3 = _sc_segment_sum(h3, src1d, dst2d, base1d, ok1d, runs1d)
    output, yp = _tc_final(P3, h3, g3d, be3d, p2d, WgT, bg2d)
    return (output, yp.reshape(1))
